# baseline (speedup 1.0000x reference)
import functools
import jax
from jax import lax
import jax.numpy as jnp
from jax.experimental import pallas as pl
from jax.experimental.pallas import tpu as pltpu
from jax.experimental.pallas import tpu_sc as plsc

E = 8
TOPK = 2
LANES = 128
BLK = 256

NC = 2
NS = 16
L = 16
NW = NC * NS


def _router_body(x_ref, gwp_ref, bias_ref, meta_ref, cnt_ref):
    T = x_ref.shape[0]
    x = x_ref[...].astype(jnp.bfloat16)
    logits = jax.lax.dot_general(x, gwp_ref[...].astype(jnp.bfloat16),
                                 (((1,), (1,)), ((), ())),
                                 preferred_element_type=jnp.float32)
    lane = jax.lax.broadcasted_iota(jnp.int32, (T, LANES), 1)
    mask8 = lane < E
    NEG = jnp.float32(-1e30)
    logits = jnp.where(mask8, logits, NEG)
    mx = jnp.max(logits, axis=1, keepdims=True)
    ex = jnp.where(mask8, jnp.exp(logits - mx), 0.0)
    scores = ex / jnp.sum(ex, axis=1, keepdims=True)
    sel = jnp.where(mask8, scores + bias_ref[0:1, :], NEG)
    v1 = jnp.max(sel, axis=1, keepdims=True)
    i1 = jnp.min(jnp.where(sel == v1, lane, 9999), axis=1, keepdims=True)
    sel2 = jnp.where(lane == i1, NEG, sel)
    v2 = jnp.max(sel2, axis=1, keepdims=True)
    i2 = jnp.min(jnp.where(sel2 == v2, lane, 9999), axis=1, keepdims=True)
    oh1 = (lane == i1).astype(jnp.float32)
    oh2 = (lane == i2).astype(jnp.float32)
    s1 = jnp.sum(scores * oh1, axis=1, keepdims=True)
    s2 = jnp.sum(scores * oh2, axis=1, keepdims=True)
    den = s1 + s2 + 1e-9
    w1 = s1 / den
    w2 = s2 / den
    ohsum = oh1 + oh2
    r_iota = jax.lax.broadcasted_iota(jnp.int32, (T, T), 0)
    c_iota = jax.lax.broadcasted_iota(jnp.int32, (T, T), 1)
    tri = (r_iota > c_iota).astype(jnp.float32)
    C = jax.lax.dot_general(tri, ohsum, (((1,), (0,)), ((), ())),
                            preferred_element_type=jnp.float32)
    r1 = jnp.sum(C * oh1, axis=1, keepdims=True)
    r2 = jnp.sum(C * oh2, axis=1, keepdims=True)
    counts = jnp.sum(ohsum, axis=0, keepdims=True)
    e1f = i1.astype(jnp.float32)
    e2f = i2.astype(jnp.float32)
    meta = (jnp.where(lane == 0, e1f, 0.0) + jnp.where(lane == 1, e2f, 0.0)
            + jnp.where(lane == 2, w1, 0.0) + jnp.where(lane == 3, w2, 0.0)
            + jnp.where(lane == 4, r1, 0.0) + jnp.where(lane == 5, r2, 0.0))
    meta_ref[...] = meta
    cnt_ref[...] = jnp.broadcast_to(counts, (8, LANES))


def _run_router(x_flat, gate_w, expert_bias):
    T = x_flat.shape[0]
    gwp = jnp.zeros((LANES, x_flat.shape[1]), jnp.float32).at[:E].set(gate_w)
    biasp = jnp.zeros((8, LANES), jnp.float32).at[0, :E].set(expert_bias)
    meta, cnt = pl.pallas_call(
        _router_body,
        out_shape=(
            jax.ShapeDtypeStruct((T, LANES), jnp.float32),
            jax.ShapeDtypeStruct((8, LANES), jnp.float32),
        ),
    )(x_flat, gwp, biasp)
    return meta, cnt


FC = 512


def _gffn_body(gid_ref, mid_ref, first_ref, tvalid_ref, starts_ref, ends_ref,
               xs_ref, gw_ref, uw_ref, dw_ref, sc_ref, out_ref):
    t = pl.program_id(0)
    c = pl.program_id(1)
    g = gid_ref[t]
    m = mid_ref[t]
    row0 = m * BLK
    rows = row0 + jax.lax.broadcasted_iota(jnp.int32, (BLK, 1), 0)
    valid = (rows >= starts_ref[g]) & (rows < ends_ref[g]) & (tvalid_ref[t] > 0)
    xs = xs_ref[...]
    gate = jax.lax.dot_general(xs, gw_ref[0], (((1,), (1,)), ((), ())),
                               preferred_element_type=jnp.float32)
    up = jax.lax.dot_general(xs, uw_ref[0], (((1,), (1,)), ((), ())),
                             preferred_element_type=jnp.float32)
    w = jnp.where(valid, sc_ref[:, 0:1], 0.0)
    h = (gate * jax.nn.sigmoid(gate)) * up * w
    contrib = jax.lax.dot_general(h, dw_ref[0], (((1,), (1,)), ((), ())),
                                  preferred_element_type=jnp.float32)

    @pl.when((first_ref[t] > 0) & (c == 0))
    def _():
        out_ref[...] = contrib

    @pl.when((first_ref[t] == 0) | (c > 0))
    def _():
        out_ref[...] += contrib


def _run_gffn(sorted_x, gate_w_e, up_w_e, down_w_e, sorted_scores,
              gid, mid, first, tvalid, starts, ends):
    R, D = sorted_x.shape
    NT = gid.shape[0]
    F = gate_w_e.shape[1]
    grid_spec = pltpu.PrefetchScalarGridSpec(
        num_scalar_prefetch=6,
        grid=(NT, F // FC),
        in_specs=[
            pl.BlockSpec((BLK, D), lambda t, c, *pref: (pref[1][t], 0)),
            pl.BlockSpec((1, FC, D), lambda t, c, *pref: (pref[0][t], c, 0)),
            pl.BlockSpec((1, FC, D), lambda t, c, *pref: (pref[0][t], c, 0)),
            pl.BlockSpec((1, D, FC), lambda t, c, *pref: (pref[0][t], 0, c)),
            pl.BlockSpec((BLK, 1), lambda t, c, *pref: (pref[1][t], 0)),
        ],
        out_specs=pl.BlockSpec((BLK, D), lambda t, c, *pref: (pref[1][t], 0)),
    )
    return pl.pallas_call(
        _gffn_body,
        grid_spec=grid_spec,
        out_shape=jax.ShapeDtypeStruct((R, D), jnp.float32),
    )(gid, mid, first, tvalid, starts, ends,
      sorted_x, gate_w_e, up_w_e, down_w_e, sorted_scores)


def _shared_body(x_ref, gw_ref, uw_ref, dw_ref, out_ref):
    x = x_ref[...]
    gate = jax.lax.dot_general(x, gw_ref[...], (((1,), (1,)), ((), ())),
                               preferred_element_type=jnp.float32)
    up = jax.lax.dot_general(x, uw_ref[...], (((1,), (1,)), ((), ())),
                             preferred_element_type=jnp.float32)
    h = (gate * jax.lax.logistic(gate)) * up
    out_ref[...] = jax.lax.dot_general(h, dw_ref[...], (((1,), (1,)), ((), ())),
                                       preferred_element_type=jnp.float32)


def _run_shared(x_flat, sgw, suw, sdw):
    T, D = x_flat.shape
    F = sgw.shape[0]
    nb = T // BLK
    return pl.pallas_call(
        _shared_body,
        grid=(nb,),
        in_specs=[
            pl.BlockSpec((BLK, D), lambda m: (m, 0)),
            pl.BlockSpec((F, D), lambda m: (0, 0)),
            pl.BlockSpec((F, D), lambda m: (0, 0)),
            pl.BlockSpec((D, F), lambda m: (0, 0)),
        ],
        out_specs=pl.BlockSpec((BLK, D), lambda m: (m, 0)),
        out_shape=jax.ShapeDtypeStruct((T, D), jnp.float32),
    )(x_flat, sgw, suw, sdw)


def _combine_body(sh_ref, g0_ref, g1_ref, out_ref):
    out_ref[...] = sh_ref[...] + g0_ref[...] + g1_ref[...]


def _run_combine(shared_out, g0, g1):
    T, D = shared_out.shape
    nb = T // BLK
    spec = pl.BlockSpec((BLK, D), lambda m: (m, 0))
    return pl.pallas_call(
        _combine_body,
        grid=(nb,),
        in_specs=[spec, spec, spec],
        out_specs=spec,
        out_shape=jax.ShapeDtypeStruct((T, D), jnp.float32),
    )(shared_out, g0, g1)


def _make_dispatch(R, D, T):
    slots = R // NW
    nch = slots // L
    mesh = plsc.VectorSubcoreMesh(core_axis_name="c", subcore_axis_name="s")

    @functools.partial(
        pl.kernel, mesh=mesh,
        out_type=(
            jax.ShapeDtypeStruct((R, D), jnp.float32),
            jax.ShapeDtypeStruct((R,), jnp.float32),
        ),
        scratch_types=[
            pltpu.VMEM((slots,), jnp.int32),
            pltpu.VMEM((slots,), jnp.int32),
            pltpu.VMEM((L, D), jnp.float32),
            pltpu.VMEM((slots,), jnp.float32),
            pltpu.SemaphoreType.DMA,
            pltpu.SemaphoreType.DMA,
        ],
    )
    def dispatch(srce_hbm, scf_hbm, x_hbm, sx_hbm, ss_hbm,
                 idxb, tokb, rows, scb, sem, sem2):
        wid = lax.axis_index("s") * NC + lax.axis_index("c")
        lo = wid * slots
        pltpu.sync_copy(srce_hbm.at[pl.ds(lo, slots)], idxb)
        for c in range(nch):
            tokb[pl.ds(c * L, L)] = lax.shift_right_logical(
                idxb[pl.ds(c * L, L)], 1)
        for c in range(nch):
            idxe = idxb[pl.ds(c * L, L)]
            pltpu.async_copy(scf_hbm.at[idxe], scb.at[pl.ds(c * L, L)],
                             sem2).wait()
        pltpu.sync_copy(scb, ss_hbm.at[pl.ds(lo, slots)])
        for c in range(nch):
            idxs = tokb[pl.ds(c * L, L)]
            pltpu.async_copy(x_hbm.at[idxs], rows, sem).wait()
            pltpu.sync_copy(rows, sx_hbm.at[pl.ds(lo + c * L, L)])

    return dispatch


def _make_combine(R, D, T):
    rpw = T // NW
    nch = rpw // L
    mesh = plsc.VectorSubcoreMesh(core_axis_name="c", subcore_axis_name="s")

    @functools.partial(
        pl.kernel, mesh=mesh,
        out_type=(
            jax.ShapeDtypeStruct((T, D), jnp.float32),
            jax.ShapeDtypeStruct((T, D), jnp.float32),
        ),
        scratch_types=[
            pltpu.VMEM((rpw,), jnp.int32),
            pltpu.VMEM((rpw,), jnp.int32),
            pltpu.VMEM((L, D), jnp.float32),
            pltpu.SemaphoreType.DMA,
        ],
    )
    def combine(p1_hbm, p2_hbm, exp_hbm, g0_hbm, g1_hbm, p1v, p2v, rows, sem):
        wid = lax.axis_index("s") * NC + lax.axis_index("c")
        base = wid * rpw
        pltpu.sync_copy(p1_hbm.at[pl.ds(base, rpw)], p1v)
        pltpu.sync_copy(p2_hbm.at[pl.ds(base, rpw)], p2v)
        for c in range(nch):
            i1 = p1v[pl.ds(c * L, L)]
            pltpu.async_copy(exp_hbm.at[i1], rows, sem).wait()
            pltpu.sync_copy(rows, g0_hbm.at[pl.ds(base + c * L, L)])
            i2 = p2v[pl.ds(c * L, L)]
            pltpu.async_copy(exp_hbm.at[i2], rows, sem).wait()
            pltpu.sync_copy(rows, g1_hbm.at[pl.ds(base + c * L, L)])

    return combine


def kernel(x, gate_w, expert_bias, shared_gate_w, shared_up_w, shared_down_w,
           expert_gate_weight, expert_up_weight, expert_down_weight):
    Bv, Sv, D = x.shape
    x_flat = x.reshape(-1, D)
    T = x_flat.shape[0]
    R = T * TOPK
    NB = R // BLK
    NT = NB + E - 1

    meta, cnt = _run_router(x_flat, gate_w, expert_bias)
    e1 = meta[:, 0].astype(jnp.int32)
    e2 = meta[:, 1].astype(jnp.int32)
    w1 = meta[:, 2]
    w2 = meta[:, 3]
    r1 = meta[:, 4].astype(jnp.int32)
    r2 = meta[:, 5].astype(jnp.int32)
    counts = cnt[0, :E].astype(jnp.int32)
    starts = jnp.concatenate([jnp.zeros((1,), jnp.int32),
                              jnp.cumsum(counts)[:-1].astype(jnp.int32)])
    ends = starts + counts
    pos1 = starts[e1] + r1
    pos2 = starts[e2] + r2

    fb = starts // BLK
    lb = (ends - 1) // BLK
    nb_e = jnp.where(counts > 0, lb - fb + 1, 0)
    tile_starts = jnp.concatenate([jnp.zeros((1,), jnp.int32),
                                   jnp.cumsum(nb_e).astype(jnp.int32)])
    total_tiles = tile_starts[E]
    tt = jnp.arange(NT, dtype=jnp.int32)
    gid_raw = jnp.sum((tt[:, None] >= tile_starts[None, 1:]).astype(jnp.int32), axis=1)
    gid = jnp.clip(gid_raw, 0, E - 1)
    mid = jnp.clip(fb[gid] + (tt - tile_starts[gid]), 0, NB - 1)
    tvalid = (tt < total_tiles).astype(jnp.int32)
    mid = jnp.where(tvalid > 0, mid, NB - 1)
    first = jnp.concatenate([jnp.ones((1,), jnp.int32),
                             (mid[1:] != mid[:-1]).astype(jnp.int32)])

    pos = jnp.stack([pos1, pos2], axis=1).reshape(-1)
    scores_flat = jnp.stack([w1, w2], axis=1).reshape(-1)
    srcentry = jnp.zeros((R,), jnp.int32).at[pos].set(
        jnp.arange(R, dtype=jnp.int32))
    sorted_x, sorted_scores = _make_dispatch(R, D, T)(srcentry, scores_flat,
                                                      x_flat)

    exp_out = _run_gffn(sorted_x, expert_gate_weight, expert_up_weight,
                        expert_down_weight, sorted_scores.reshape(R, 1),
                        gid, mid, first, tvalid, starts, ends)

    shared_out = _run_shared(x_flat, shared_gate_w, shared_up_w,
                             shared_down_w)

    g0, g1 = _make_combine(R, D, T)(pos1, pos2, exp_out)
    out = _run_combine(shared_out, g0, g1)
    return out.reshape(Bv, Sv, D)


# speedup vs baseline: 1.1957x; 1.1957x over previous
import functools
import jax
from jax import lax
import jax.numpy as jnp
from jax.experimental import pallas as pl
from jax.experimental.pallas import tpu as pltpu
from jax.experimental.pallas import tpu_sc as plsc

E = 8
TOPK = 2
LANES = 128
BLK = 256

NC = 2
NS = 16
L = 16
NW = NC * NS


def _router_body(x_ref, gwp_ref, bias_ref, meta_ref, cnt_ref):
    T = x_ref.shape[0]
    x = x_ref[...].astype(jnp.bfloat16)
    logits = jax.lax.dot_general(x, gwp_ref[...].astype(jnp.bfloat16),
                                 (((1,), (1,)), ((), ())),
                                 preferred_element_type=jnp.float32)
    lane = jax.lax.broadcasted_iota(jnp.int32, (T, LANES), 1)
    mask8 = lane < E
    NEG = jnp.float32(-1e30)
    logits = jnp.where(mask8, logits, NEG)
    mx = jnp.max(logits, axis=1, keepdims=True)
    ex = jnp.where(mask8, jnp.exp(logits - mx), 0.0)
    scores = ex / jnp.sum(ex, axis=1, keepdims=True)
    sel = jnp.where(mask8, scores + bias_ref[0:1, :], NEG)
    v1 = jnp.max(sel, axis=1, keepdims=True)
    i1 = jnp.min(jnp.where(sel == v1, lane, 9999), axis=1, keepdims=True)
    sel2 = jnp.where(lane == i1, NEG, sel)
    v2 = jnp.max(sel2, axis=1, keepdims=True)
    i2 = jnp.min(jnp.where(sel2 == v2, lane, 9999), axis=1, keepdims=True)
    oh1 = (lane == i1).astype(jnp.float32)
    oh2 = (lane == i2).astype(jnp.float32)
    s1 = jnp.sum(scores * oh1, axis=1, keepdims=True)
    s2 = jnp.sum(scores * oh2, axis=1, keepdims=True)
    den = s1 + s2 + 1e-9
    w1 = s1 / den
    w2 = s2 / den
    ohsum = oh1 + oh2
    r_iota = jax.lax.broadcasted_iota(jnp.int32, (T, T), 0)
    c_iota = jax.lax.broadcasted_iota(jnp.int32, (T, T), 1)
    tri = (r_iota > c_iota).astype(jnp.float32)
    C = jax.lax.dot_general(tri, ohsum, (((1,), (0,)), ((), ())),
                            preferred_element_type=jnp.float32)
    r1 = jnp.sum(C * oh1, axis=1, keepdims=True)
    r2 = jnp.sum(C * oh2, axis=1, keepdims=True)
    counts = jnp.sum(ohsum, axis=0, keepdims=True)
    e1f = i1.astype(jnp.float32)
    e2f = i2.astype(jnp.float32)
    meta = (jnp.where(lane == 0, e1f, 0.0) + jnp.where(lane == 1, e2f, 0.0)
            + jnp.where(lane == 2, w1, 0.0) + jnp.where(lane == 3, w2, 0.0)
            + jnp.where(lane == 4, r1, 0.0) + jnp.where(lane == 5, r2, 0.0))
    meta_ref[...] = meta
    cnt_ref[...] = jnp.broadcast_to(counts, (8, LANES))


def _run_router(x_flat, gate_w, expert_bias):
    T = x_flat.shape[0]
    gwp = jnp.zeros((LANES, x_flat.shape[1]), jnp.float32).at[:E].set(gate_w)
    biasp = jnp.zeros((8, LANES), jnp.float32).at[0, :E].set(expert_bias)
    meta, cnt = pl.pallas_call(
        _router_body,
        out_shape=(
            jax.ShapeDtypeStruct((T, LANES), jnp.float32),
            jax.ShapeDtypeStruct((8, LANES), jnp.float32),
        ),
    )(x_flat, gwp, biasp)
    return meta, cnt


FC = 512


def _gffn_body(gid_ref, mid_ref, first_ref, tvalid_ref, starts_ref, ends_ref,
               xs_ref, gw_ref, uw_ref, dw_ref, sc_ref, out_ref):
    t = pl.program_id(0)
    c = pl.program_id(1)
    g = gid_ref[t]
    m = mid_ref[t]
    row0 = m * BLK
    rows = row0 + jax.lax.broadcasted_iota(jnp.int32, (BLK, 1), 0)
    valid = (rows >= starts_ref[g]) & (rows < ends_ref[g]) & (tvalid_ref[t] > 0)
    xs = xs_ref[...]
    gate = jax.lax.dot_general(xs, gw_ref[0], (((1,), (1,)), ((), ())),
                               preferred_element_type=jnp.float32)
    up = jax.lax.dot_general(xs, uw_ref[0], (((1,), (1,)), ((), ())),
                             preferred_element_type=jnp.float32)
    w = jnp.where(valid, sc_ref[:, 0:1], 0.0)
    h = (gate * jax.nn.sigmoid(gate)) * up * w
    contrib = jax.lax.dot_general(h, dw_ref[0], (((1,), (1,)), ((), ())),
                                  preferred_element_type=jnp.float32)

    @pl.when((first_ref[t] > 0) & (c == 0))
    def _():
        out_ref[...] = contrib

    @pl.when((first_ref[t] == 0) | (c > 0))
    def _():
        out_ref[...] += contrib


def _run_gffn(sorted_x, gate_w_e, up_w_e, down_w_e, sorted_scores,
              gid, mid, first, tvalid, starts, ends):
    R, D = sorted_x.shape
    NT = gid.shape[0]
    F = gate_w_e.shape[1]
    grid_spec = pltpu.PrefetchScalarGridSpec(
        num_scalar_prefetch=6,
        grid=(NT, 1),
        in_specs=[
            pl.BlockSpec((BLK, D), lambda t, c, *pref: (pref[1][t], 0)),
            pl.BlockSpec((1, F, D), lambda t, c, *pref: (pref[0][t], 0, 0)),
            pl.BlockSpec((1, F, D), lambda t, c, *pref: (pref[0][t], 0, 0)),
            pl.BlockSpec((1, D, F), lambda t, c, *pref: (pref[0][t], 0, 0)),
            pl.BlockSpec((BLK, 1), lambda t, c, *pref: (pref[1][t], 0)),
        ],
        out_specs=pl.BlockSpec((BLK, D), lambda t, c, *pref: (pref[1][t], 0)),
    )
    return pl.pallas_call(
        _gffn_body,
        grid_spec=grid_spec,
        out_shape=jax.ShapeDtypeStruct((R, D), jnp.float32),
    )(gid, mid, first, tvalid, starts, ends,
      sorted_x, gate_w_e, up_w_e, down_w_e, sorted_scores)


def _shared_body(x_ref, gw_ref, uw_ref, dw_ref, out_ref):
    x = x_ref[...]
    gate = jax.lax.dot_general(x, gw_ref[...], (((1,), (1,)), ((), ())),
                               preferred_element_type=jnp.float32)
    up = jax.lax.dot_general(x, uw_ref[...], (((1,), (1,)), ((), ())),
                             preferred_element_type=jnp.float32)
    h = (gate * jax.lax.logistic(gate)) * up
    out_ref[...] = jax.lax.dot_general(h, dw_ref[...], (((1,), (1,)), ((), ())),
                                       preferred_element_type=jnp.float32)


def _run_shared(x_flat, sgw, suw, sdw):
    T, D = x_flat.shape
    F = sgw.shape[0]
    nb = T // BLK
    return pl.pallas_call(
        _shared_body,
        grid=(nb,),
        in_specs=[
            pl.BlockSpec((BLK, D), lambda m: (m, 0)),
            pl.BlockSpec((F, D), lambda m: (0, 0)),
            pl.BlockSpec((F, D), lambda m: (0, 0)),
            pl.BlockSpec((D, F), lambda m: (0, 0)),
        ],
        out_specs=pl.BlockSpec((BLK, D), lambda m: (m, 0)),
        out_shape=jax.ShapeDtypeStruct((T, D), jnp.float32),
    )(x_flat, sgw, suw, sdw)


def _combine_body(sh_ref, g0_ref, g1_ref, out_ref):
    out_ref[...] = sh_ref[...] + g0_ref[...] + g1_ref[...]


def _run_combine(shared_out, g0, g1):
    T, D = shared_out.shape
    nb = T // BLK
    spec = pl.BlockSpec((BLK, D), lambda m: (m, 0))
    return pl.pallas_call(
        _combine_body,
        grid=(nb,),
        in_specs=[spec, spec, spec],
        out_specs=spec,
        out_shape=jax.ShapeDtypeStruct((T, D), jnp.float32),
    )(shared_out, g0, g1)


def _make_dispatch(R, D, T):
    slots = R // NW
    nch = slots // L
    mesh = plsc.VectorSubcoreMesh(core_axis_name="c", subcore_axis_name="s")

    @functools.partial(
        pl.kernel, mesh=mesh,
        out_type=(
            jax.ShapeDtypeStruct((R, D), jnp.float32),
            jax.ShapeDtypeStruct((R,), jnp.float32),
        ),
        scratch_types=[
            pltpu.VMEM((slots,), jnp.int32),
            pltpu.VMEM((slots,), jnp.int32),
            pltpu.VMEM((L, D), jnp.float32),
            pltpu.VMEM((slots,), jnp.float32),
            pltpu.SemaphoreType.DMA,
            pltpu.SemaphoreType.DMA,
        ],
    )
    def dispatch(srce_hbm, scf_hbm, x_hbm, sx_hbm, ss_hbm,
                 idxb, tokb, rows, scb, sem, sem2):
        wid = lax.axis_index("s") * NC + lax.axis_index("c")
        lo = wid * slots
        pltpu.sync_copy(srce_hbm.at[pl.ds(lo, slots)], idxb)
        for c in range(nch):
            tokb[pl.ds(c * L, L)] = lax.shift_right_logical(
                idxb[pl.ds(c * L, L)], 1)
        for c in range(nch):
            idxe = idxb[pl.ds(c * L, L)]
            pltpu.async_copy(scf_hbm.at[idxe], scb.at[pl.ds(c * L, L)],
                             sem2).wait()
        pltpu.sync_copy(scb, ss_hbm.at[pl.ds(lo, slots)])
        for c in range(nch):
            idxs = tokb[pl.ds(c * L, L)]
            pltpu.async_copy(x_hbm.at[idxs], rows, sem).wait()
            pltpu.sync_copy(rows, sx_hbm.at[pl.ds(lo + c * L, L)])

    return dispatch


def _make_combine(R, D, T):
    rpw = T // NW
    nch = rpw // L
    mesh = plsc.VectorSubcoreMesh(core_axis_name="c", subcore_axis_name="s")

    @functools.partial(
        pl.kernel, mesh=mesh,
        out_type=(
            jax.ShapeDtypeStruct((T, D), jnp.float32),
            jax.ShapeDtypeStruct((T, D), jnp.float32),
        ),
        scratch_types=[
            pltpu.VMEM((rpw,), jnp.int32),
            pltpu.VMEM((rpw,), jnp.int32),
            pltpu.VMEM((L, D), jnp.float32),
            pltpu.SemaphoreType.DMA,
        ],
    )
    def combine(p1_hbm, p2_hbm, exp_hbm, g0_hbm, g1_hbm, p1v, p2v, rows, sem):
        wid = lax.axis_index("s") * NC + lax.axis_index("c")
        base = wid * rpw
        pltpu.sync_copy(p1_hbm.at[pl.ds(base, rpw)], p1v)
        pltpu.sync_copy(p2_hbm.at[pl.ds(base, rpw)], p2v)
        for c in range(nch):
            i1 = p1v[pl.ds(c * L, L)]
            pltpu.async_copy(exp_hbm.at[i1], rows, sem).wait()
            pltpu.sync_copy(rows, g0_hbm.at[pl.ds(base + c * L, L)])
            i2 = p2v[pl.ds(c * L, L)]
            pltpu.async_copy(exp_hbm.at[i2], rows, sem).wait()
            pltpu.sync_copy(rows, g1_hbm.at[pl.ds(base + c * L, L)])

    return combine


def kernel(x, gate_w, expert_bias, shared_gate_w, shared_up_w, shared_down_w,
           expert_gate_weight, expert_up_weight, expert_down_weight):
    Bv, Sv, D = x.shape
    x_flat = x.reshape(-1, D)
    T = x_flat.shape[0]
    R = T * TOPK
    NB = R // BLK
    NT = NB + E - 1

    meta, cnt = _run_router(x_flat, gate_w, expert_bias)
    e1 = meta[:, 0].astype(jnp.int32)
    e2 = meta[:, 1].astype(jnp.int32)
    w1 = meta[:, 2]
    w2 = meta[:, 3]
    r1 = meta[:, 4].astype(jnp.int32)
    r2 = meta[:, 5].astype(jnp.int32)
    counts = cnt[0, :E].astype(jnp.int32)
    starts = jnp.concatenate([jnp.zeros((1,), jnp.int32),
                              jnp.cumsum(counts)[:-1].astype(jnp.int32)])
    ends = starts + counts
    pos1 = starts[e1] + r1
    pos2 = starts[e2] + r2

    fb = starts // BLK
    lb = (ends - 1) // BLK
    nb_e = jnp.where(counts > 0, lb - fb + 1, 0)
    tile_starts = jnp.concatenate([jnp.zeros((1,), jnp.int32),
                                   jnp.cumsum(nb_e).astype(jnp.int32)])
    total_tiles = tile_starts[E]
    tt = jnp.arange(NT, dtype=jnp.int32)
    gid_raw = jnp.sum((tt[:, None] >= tile_starts[None, 1:]).astype(jnp.int32), axis=1)
    gid = jnp.clip(gid_raw, 0, E - 1)
    mid = jnp.clip(fb[gid] + (tt - tile_starts[gid]), 0, NB - 1)
    tvalid = (tt < total_tiles).astype(jnp.int32)
    mid = jnp.where(tvalid > 0, mid, NB - 1)
    first = jnp.concatenate([jnp.ones((1,), jnp.int32),
                             (mid[1:] != mid[:-1]).astype(jnp.int32)])

    pos = jnp.stack([pos1, pos2], axis=1).reshape(-1)
    scores_flat = jnp.stack([w1, w2], axis=1).reshape(-1)
    srcentry = jnp.zeros((R,), jnp.int32).at[pos].set(
        jnp.arange(R, dtype=jnp.int32))
    sorted_x, sorted_scores = _make_dispatch(R, D, T)(srcentry, scores_flat,
                                                      x_flat)

    exp_out = _run_gffn(sorted_x, expert_gate_weight, expert_up_weight,
                        expert_down_weight, sorted_scores.reshape(R, 1),
                        gid, mid, first, tvalid, starts, ends)

    shared_out = _run_shared(x_flat, shared_gate_w, shared_up_w,
                             shared_down_w)

    g0, g1 = _make_combine(R, D, T)(pos1, pos2, exp_out)
    out = _run_combine(shared_out, g0, g1)
    return out.reshape(Bv, Sv, D)
